# Initial kernel scaffold; baseline (speedup 1.0000x reference)
import jax
import jax.numpy as jnp
from jax.experimental import pallas as pl


def kernel(inputs, z):
    raise NotImplementedError("write your pallas kernel here")


# speedup vs baseline: 1.0884x; 1.0884x over previous
import functools

import jax
import jax.numpy as jnp
from jax import lax
from jax.experimental import pallas as pl
from jax.experimental.pallas import tpu as pltpu
from jax.experimental.pallas import tpu_sc as plsc

N_STIMULI = 1000000
N_DIM = 32
BATCH = 16384
HIST = 50

NUM_WORKERS = 32
TRIALS_PER_WORKER = BATCH // NUM_WORKERS
CHUNK_T = 16
NUM_CHUNKS = TRIALS_PER_WORKER // CHUNK_T
ROWS = CHUNK_T * HIST
SUB = 80
NSUB = ROWS // SUB

_mesh = plsc.VectorSubcoreMesh(
    core_axis_name="c", subcore_axis_name="s", num_cores=2, num_subcores=16
)


@functools.partial(
    pl.kernel,
    out_type=jax.ShapeDtypeStruct((BATCH, N_DIM, HIST), jnp.float32),
    mesh=_mesh,
    scratch_types=[
        pltpu.VMEM((ROWS,), jnp.int32),
        pltpu.VMEM((ROWS, N_DIM), jnp.float32),
        pltpu.VMEM((CHUNK_T, N_DIM, HIST), jnp.float32),
        pltpu.SemaphoreType.DMA,
    ],
    compiler_params=pltpu.CompilerParams(
        needs_layout_passes=False, use_tc_tiling_on_sc=False
    ),
)
def _sc_embed(z_hbm, idx_hbm, out_hbm, idx_v, buf, obuf, sem_g):
    wid = lax.axis_index("s") * 2 + lax.axis_index("c")
    iota = lax.iota(jnp.int32, 16)

    def chunk_body(c, carry):
        base_t = wid * TRIALS_PER_WORKER + c * CHUNK_T
        pltpu.sync_copy(idx_hbm.at[pl.ds(base_t * HIST, ROWS)], idx_v)
        copies = []
        for k in range(NSUB):
            copies.append(
                pltpu.async_copy(
                    z_hbm.at[idx_v.at[pl.ds(k * SUB, SUB)]],
                    buf.at[pl.ds(k * SUB, SUB)],
                    sem_g,
                )
            )
        for cp in copies:
            cp.wait()

        def t_body(t, carry2):
            row0 = t * HIST
            for d in range(N_DIM):
                dcol = jnp.full((16,), d, jnp.int32)
                for h0 in (0, 16, 32, 34):
                    rows = row0 + h0 + iota
                    v = plsc.load_gather(buf, [rows, dcol])
                    obuf[t, d, pl.ds(h0, 16)] = v
            return carry2

        lax.fori_loop(0, CHUNK_T, t_body, 0)
        pltpu.sync_copy(obuf, out_hbm.at[pl.ds(base_t, CHUNK_T)])
        return carry

    lax.fori_loop(0, NUM_CHUNKS, chunk_body, 0)


def kernel(inputs, z):
    return _sc_embed(z, inputs.reshape(-1))


# speedup vs baseline: 1.1427x; 1.0499x over previous
import functools

import jax
import jax.numpy as jnp
from jax import lax
from jax.experimental import pallas as pl
from jax.experimental.pallas import tpu as pltpu
from jax.experimental.pallas import tpu_sc as plsc

N_STIMULI = 1000000
N_DIM = 32
BATCH = 16384
HIST = 50

NUM_WORKERS = 32
TRIALS_PER_WORKER = BATCH // NUM_WORKERS
CHUNK_T = 16
NUM_CHUNKS = TRIALS_PER_WORKER // CHUNK_T
NUM_PAIRS = NUM_CHUNKS // 2
ROWS = CHUNK_T * HIST
SUB = 80
NSUB = ROWS // SUB

_mesh = plsc.VectorSubcoreMesh(
    core_axis_name="c", subcore_axis_name="s", num_cores=2, num_subcores=16
)


@functools.partial(
    pl.kernel,
    out_type=jax.ShapeDtypeStruct((BATCH, N_DIM, HIST), jnp.float32),
    mesh=_mesh,
    scratch_types=[
        pltpu.VMEM((ROWS,), jnp.int32),
        pltpu.VMEM((ROWS,), jnp.int32),
        pltpu.VMEM((ROWS, N_DIM), jnp.float32),
        pltpu.VMEM((ROWS, N_DIM), jnp.float32),
        pltpu.VMEM((CHUNK_T, N_DIM, HIST), jnp.float32),
        pltpu.VMEM((CHUNK_T, N_DIM, HIST), jnp.float32),
        pltpu.SemaphoreType.DMA,
        pltpu.SemaphoreType.DMA,
        pltpu.SemaphoreType.DMA,
        pltpu.SemaphoreType.DMA,
    ],
    compiler_params=pltpu.CompilerParams(
        needs_layout_passes=False, use_tc_tiling_on_sc=False
    ),
)
def _sc_embed(z_hbm, idx_hbm, out_hbm, idx0, idx1, buf0, buf1, ob0, ob1,
              semg0, semg1, semo0, semo1):
    wid = lax.axis_index("s") * 2 + lax.axis_index("c")
    tbase = wid * TRIALS_PER_WORKER
    iota = lax.iota(jnp.int32, 16)

    def fire_gathers(idx_v, buf, sem):
        for k in range(NSUB):
            pltpu.async_copy(
                z_hbm.at[idx_v.at[pl.ds(k * SUB, SUB)]],
                buf.at[pl.ds(k * SUB, SUB)],
                sem,
            )

    def drain_gathers(idx_v, buf, sem):
        for k in range(NSUB):
            pltpu.make_async_copy(
                z_hbm.at[idx_v.at[pl.ds(k * SUB, SUB)]],
                buf.at[pl.ds(k * SUB, SUB)],
                sem,
            ).wait()

    def transpose(buf, ob):
        def t_body(t, carry):
            row0 = t * HIST
            for d in range(N_DIM):
                dcol = jnp.full((16,), d, jnp.int32)
                for h0 in (0, 16, 32, 34):
                    rows = row0 + h0 + iota
                    v = plsc.load_gather(buf, [rows, dcol])
                    ob[t, d, pl.ds(h0, 16)] = v
            return carry

        lax.fori_loop(0, CHUNK_T, t_body, 0)

    pltpu.sync_copy(idx_hbm.at[pl.ds(tbase * HIST, ROWS)], idx0)
    fire_gathers(idx0, buf0, semg0)

    def body(i, carry):
        b0 = tbase + (2 * i) * CHUNK_T
        b1 = tbase + (2 * i + 1) * CHUNK_T

        pltpu.sync_copy(idx_hbm.at[pl.ds(b1 * HIST, ROWS)], idx1)
        fire_gathers(idx1, buf1, semg1)

        drain_gathers(idx0, buf0, semg0)

        @pl.when(i > 0)
        def _():
            pltpu.make_async_copy(
                ob0, out_hbm.at[pl.ds(b0, CHUNK_T)], semo0
            ).wait()

        transpose(buf0, ob0)
        pltpu.async_copy(ob0, out_hbm.at[pl.ds(b0, CHUNK_T)], semo0)

        @pl.when(i + 1 < NUM_PAIRS)
        def _():
            b2 = tbase + (2 * i + 2) * CHUNK_T
            pltpu.sync_copy(idx_hbm.at[pl.ds(b2 * HIST, ROWS)], idx0)
            fire_gathers(idx0, buf0, semg0)

        drain_gathers(idx1, buf1, semg1)

        @pl.when(i > 0)
        def _():
            pltpu.make_async_copy(
                ob1, out_hbm.at[pl.ds(b1, CHUNK_T)], semo1
            ).wait()

        transpose(buf1, ob1)
        pltpu.async_copy(ob1, out_hbm.at[pl.ds(b1, CHUNK_T)], semo1)
        return carry

    lax.fori_loop(0, NUM_PAIRS, body, 0)

    pltpu.make_async_copy(ob0, out_hbm.at[pl.ds(tbase, CHUNK_T)], semo0).wait()
    pltpu.make_async_copy(ob1, out_hbm.at[pl.ds(tbase, CHUNK_T)], semo1).wait()


def kernel(inputs, z):
    return _sc_embed(z, inputs.reshape(-1))
